# baseline (speedup 1.0000x reference)
import functools

import jax
import jax.numpy as jnp
from jax.experimental import pallas as pl
from jax.experimental.pallas import tpu as pltpu

_K = 32


def _lrelu(t):
    return jnp.where(t >= 0, t, 0.1 * t)


def _det_body(x_ref, w1_ref, b1_ref, w2_ref, b2_ref,
              vals_ref, idx_ref, gidx_ref, p_ref, *, nsteps, batch, seq):
    i = pl.program_id(0)
    rb = x_ref.shape[0]
    h = _lrelu(jnp.dot(x_ref[...], w1_ref[...],
                       preferred_element_type=jnp.float32) + b1_ref[...])
    logit = jax.lax.dot_general(w2_ref[...], h, (((1,), (1,)), ((), ())),
                                preferred_element_type=jnp.float32)
    p_ref[pl.ds(i, 1), :] = jax.nn.sigmoid(logit + b2_ref[...])

    @pl.when(i == nsteps - 1)
    def _():
        rows_per_b = seq // rb
        kcol = jax.lax.broadcasted_iota(jnp.int32, (1, _K), 1)
        for b in range(batch):
            p0 = p_ref[b * rows_per_b:(b + 1) * rows_per_b, :]
            fid = (jax.lax.broadcasted_iota(jnp.int32, (rows_per_b, rb), 0)
                   * rb
                   + jax.lax.broadcasted_iota(jnp.int32, (rows_per_b, rb), 1))

            def body(j, carry):
                p, vals, idxs = carry
                m = jnp.max(p)
                s = jnp.min(jnp.where(p == m, fid, seq))
                vals = jnp.where(kcol == j, m, vals)
                idxs = jnp.where(kcol == j, s, idxs)
                p = jnp.where(fid == s, -1.0, p)
                return p, vals, idxs

            _, vals, idxs = jax.lax.fori_loop(
                0, _K, body,
                (p0, jnp.zeros((1, _K), jnp.float32),
                 jnp.zeros((1, _K), jnp.int32)))
            vals_ref[b:b + 1, :] = vals
            idx_ref[b:b + 1, :] = idxs
            gidx_ref[b:b + 1, :] = idxs + b * seq


def _detector_topk(x2d, W1, b1, W2, b2, rb, batch, seq):
    M, H = x2d.shape
    nsteps = M // rb
    body = functools.partial(_det_body, nsteps=nsteps, batch=batch, seq=seq)
    return pl.pallas_call(
        body,
        grid=(nsteps,),
        in_specs=[
            pl.BlockSpec((rb, H), lambda i: (i, 0)),
            pl.BlockSpec((H, H), lambda i: (0, 0)),
            pl.BlockSpec((1, H), lambda i: (0, 0)),
            pl.BlockSpec((1, H), lambda i: (0, 0)),
            pl.BlockSpec((1, 1), lambda i: (0, 0)),
        ],
        out_specs=(
            pl.BlockSpec((batch, _K), lambda i: (0, 0)),
            pl.BlockSpec((batch, _K), lambda i: (0, 0)),
            pl.BlockSpec((batch, _K), lambda i: (0, 0)),
        ),
        out_shape=(
            jax.ShapeDtypeStruct((batch, _K), jnp.float32),
            jax.ShapeDtypeStruct((batch, _K), jnp.int32),
            jax.ShapeDtypeStruct((batch, _K), jnp.int32),
        ),
        scratch_shapes=[pltpu.VMEM((nsteps, rb), jnp.float32)],
    )(x2d, W1, b1.reshape(1, H), W2.reshape(1, H), b2.reshape(1, 1))


def _pn_body(gidx_ref, x_ref, w1_ref, b1_ref, w2_ref, b2_ref, idw_ref,
             idb_ref, gw_ref, gb_ref, tv_ref, ti_ref,
             tim_ref, ids_ref, g_ref, xg_ref, acc_ref, sem, *, nsteps):
    j = pl.program_id(0)
    R = xg_ref.shape[0]

    @pl.when(j == 0)
    def _():
        for r in range(R):
            pltpu.make_async_copy(x_ref.at[pl.ds(gidx_ref[r], 1)],
                                  xg_ref.at[pl.ds(r, 1)], sem).start()
        for r in range(R):
            pltpu.make_async_copy(x_ref.at[pl.ds(gidx_ref[r], 1)],
                                  xg_ref.at[pl.ds(r, 1)], sem).wait()

    f1 = _lrelu(jnp.dot(xg_ref[...], w1_ref[...],
                        preferred_element_type=jnp.float32) + b1_ref[...])
    part = jnp.dot(f1, w2_ref[...], preferred_element_type=jnp.float32)

    @pl.when(j == 0)
    def _():
        acc_ref[...] = part

    @pl.when(j > 0)
    def _():
        acc_ref[...] += part

    @pl.when(j == nsteps - 1)
    def _():
        R = acc_ref.shape[0]
        N = idw_ref.shape[1]
        f2 = _lrelu(acc_ref[...] + b2_ref[...])
        logits = jnp.dot(f2, idw_ref[...],
                         preferred_element_type=jnp.float32) + idb_ref[...]
        m = jnp.max(logits, axis=1, keepdims=True)
        ncol = jax.lax.broadcasted_iota(jnp.int32, (R, N), 1)
        amax = jnp.min(jnp.where(logits == m, ncol, N), axis=1, keepdims=True)
        gl = jnp.sum(f2 * gw_ref[...], axis=1, keepdims=True) + gb_ref[...]
        gains = jax.nn.sigmoid(gl)
        mask = tv_ref[...] > 0.5
        tim_ref[...] = jnp.where(mask, ti_ref[...].astype(jnp.float32) * 0.01,
                                 0.0)
        ids_ref[...] = jnp.where(mask, amax, 0)
        g_ref[...] = jnp.where(mask, gains, 0.0)


def _param_net(x2d, gidx, W1, b1, W2, b2, idW, idb, gW, gb, tvals, tidx, cb):
    H = x2d.shape[1]
    R = gidx.shape[0]
    N = idW.shape[1]
    nsteps = H // cb
    body = functools.partial(_pn_body, nsteps=nsteps)
    grid_spec = pltpu.PrefetchScalarGridSpec(
        num_scalar_prefetch=1,
        grid=(nsteps,),
        in_specs=[
            pl.BlockSpec(memory_space=pl.ANY),
            pl.BlockSpec((H, cb), lambda j, gi: (0, j)),
            pl.BlockSpec((1, cb), lambda j, gi: (0, j)),
            pl.BlockSpec((cb, H), lambda j, gi: (j, 0)),
            pl.BlockSpec((1, H), lambda j, gi: (0, 0)),
            pl.BlockSpec((H, N), lambda j, gi: (0, 0)),
            pl.BlockSpec((1, N), lambda j, gi: (0, 0)),
            pl.BlockSpec((1, H), lambda j, gi: (0, 0)),
            pl.BlockSpec((1, 1), lambda j, gi: (0, 0)),
            pl.BlockSpec((R, 1), lambda j, gi: (0, 0)),
            pl.BlockSpec((R, 1), lambda j, gi: (0, 0)),
        ],
        out_specs=(
            pl.BlockSpec((R, 1), lambda j, gi: (0, 0)),
            pl.BlockSpec((R, 1), lambda j, gi: (0, 0)),
            pl.BlockSpec((R, 1), lambda j, gi: (0, 0)),
        ),
        scratch_shapes=[
            pltpu.VMEM((R, H), jnp.float32),
            pltpu.VMEM((R, H), jnp.float32),
            pltpu.SemaphoreType.DMA,
        ],
    )
    return pl.pallas_call(
        body,
        grid_spec=grid_spec,
        out_shape=(
            jax.ShapeDtypeStruct((R, 1), jnp.float32),
            jax.ShapeDtypeStruct((R, 1), jnp.int32),
            jax.ShapeDtypeStruct((R, 1), jnp.float32),
        ),
    )(gidx, x2d, W1, b1.reshape(1, H), W2, b2.reshape(1, H), idW,
      idb.reshape(1, N), gW.reshape(1, H), gb.reshape(1, 1), tvals, tidx)


def kernel(x, det_W1, det_b1, det_W2, det_b2, pn_W1, pn_b1, pn_W2, pn_b2,
           id_W, id_b, g_W, g_b):
    B, S, H = x.shape
    x2d = x.reshape(B * S, H)

    tvals, tidx, gidx = _detector_topk(x2d, det_W1, det_b1, det_W2, det_b2,
                                       rb=512, batch=B, seq=S)
    tim, ids, gains = _param_net(
        x2d, gidx.reshape(-1), pn_W1, pn_b1, pn_W2, pn_b2, id_W, id_b,
        g_W, g_b, tvals.reshape(B * _K, 1), tidx.reshape(B * _K, 1), cb=512)
    return (tim.reshape(B, _K), ids.reshape(B, _K), gains.reshape(B, _K))


# speedup vs baseline: 1.0157x; 1.0157x over previous
import functools

import jax
import jax.numpy as jnp
from jax.experimental import pallas as pl
from jax.experimental.pallas import tpu as pltpu

_K = 32


def _lrelu(t):
    return jnp.where(t >= 0, t, 0.1 * t)


def _mega_body(x_ref, xany_ref, w1_ref, b1_ref, w2_ref, b2_ref,
               pnw1_any, pnb1_ref, pnw2_any, pnb2_ref,
               idw_ref, idb_ref, gw_ref, gb_ref,
               tim_ref, ids_ref, g_ref,
               p_ref, w1s_ref, w2s_ref, xg_ref, semw, semg,
               *, nsteps, batch, seq, rb):
    i = pl.program_id(0)

    @pl.when(i == 0)
    def _():
        pltpu.make_async_copy(pnw1_any, w1s_ref, semw).start()
        pltpu.make_async_copy(pnw2_any, w2s_ref, semw).start()

    @pl.when(i < nsteps)
    def _():
        h = _lrelu(jnp.dot(x_ref[...], w1_ref[...],
                           preferred_element_type=jnp.float32) + b1_ref[...])
        logit = jax.lax.dot_general(w2_ref[...], h, (((1,), (1,)), ((), ())),
                                    preferred_element_type=jnp.float32)
        p_ref[pl.ds(i, 1), :] = jax.nn.sigmoid(logit + b2_ref[...])

    @pl.when(i == nsteps)
    def _():
        rows_per_b = seq // rb
        R = batch * _K
        fid = (jax.lax.broadcasted_iota(jnp.int32, (rows_per_b, rb), 0) * rb
               + jax.lax.broadcasted_iota(jnp.int32, (rows_per_b, rb), 1))
        krow = jax.lax.broadcasted_iota(jnp.int32, (_K, 1), 0)

        tv_list, ti_list, gidx_list = [], [], []
        for b in range(batch):
            p0 = p_ref[b * rows_per_b:(b + 1) * rows_per_b, :]

            def body(j, carry):
                p, vals, idxs = carry
                m = jnp.max(p)
                s = jnp.min(jnp.where(p == m, fid, seq))
                vals = jnp.where(krow == j, m, vals)
                idxs = jnp.where(krow == j, s, idxs)
                p = jnp.where(fid == s, -1.0, p)
                return p, vals, idxs

            _, vals, idxs = jax.lax.fori_loop(
                0, _K, body,
                (p0, jnp.zeros((_K, 1), jnp.float32),
                 jnp.zeros((_K, 1), jnp.int32)))
            tv_list.append(vals)
            ti_list.append(idxs)
            gidx_list.append(idxs + b * seq)

        copies = []
        for b in range(batch):
            gidx = gidx_list[b]
            for j in range(_K):
                s = jnp.max(jnp.where(krow == j, gidx, 0))
                r = b * _K + j
                cp = pltpu.make_async_copy(xany_ref.at[pl.ds(s, 1)],
                                           xg_ref.at[pl.ds(r, 1)], semg)
                cp.start()
                copies.append(cp)
        for cp in copies:
            cp.wait()
        pltpu.make_async_copy(pnw1_any, w1s_ref, semw).wait()
        pltpu.make_async_copy(pnw2_any, w2s_ref, semw).wait()

        N = idw_ref.shape[1]
        f1 = _lrelu(jnp.dot(xg_ref[...], w1s_ref[...],
                            preferred_element_type=jnp.float32)
                    + pnb1_ref[...])
        f2 = _lrelu(jnp.dot(f1, w2s_ref[...],
                            preferred_element_type=jnp.float32)
                    + pnb2_ref[...])
        logits = jnp.dot(f2, idw_ref[...],
                         preferred_element_type=jnp.float32) + idb_ref[...]
        m = jnp.max(logits, axis=1, keepdims=True)
        ncol = jax.lax.broadcasted_iota(jnp.int32, (R, N), 1)
        amax = jnp.min(jnp.where(logits == m, ncol, N), axis=1, keepdims=True)
        gl = jnp.sum(f2 * gw_ref[...], axis=1, keepdims=True) + gb_ref[...]
        gains = jax.nn.sigmoid(gl)

        tv = jnp.concatenate(tv_list, axis=0)
        ti = jnp.concatenate(ti_list, axis=0)
        mask = tv > 0.5
        tim_ref[...] = jnp.where(mask, ti.astype(jnp.float32) * 0.01, 0.0)
        ids_ref[...] = jnp.where(mask, amax, 0)
        g_ref[...] = jnp.where(mask, gains, 0.0)


def kernel(x, det_W1, det_b1, det_W2, det_b2, pn_W1, pn_b1, pn_W2, pn_b2,
           id_W, id_b, g_W, g_b):
    B, S, H = x.shape
    N = id_W.shape[1]
    M = B * S
    R = B * _K
    rb = 256
    nsteps = M // rb
    x2d = x.reshape(M, H)

    body = functools.partial(_mega_body, nsteps=nsteps, batch=B, seq=S, rb=rb)
    tim, ids, gains = pl.pallas_call(
        body,
        grid=(nsteps + 1,),
        in_specs=[
            pl.BlockSpec((rb, H), lambda i: (jnp.minimum(i, nsteps - 1), 0)),
            pl.BlockSpec(memory_space=pl.ANY),
            pl.BlockSpec((H, H), lambda i: (0, 0)),
            pl.BlockSpec((1, H), lambda i: (0, 0)),
            pl.BlockSpec((1, H), lambda i: (0, 0)),
            pl.BlockSpec((1, 1), lambda i: (0, 0)),
            pl.BlockSpec(memory_space=pl.ANY),
            pl.BlockSpec((1, H), lambda i: (0, 0)),
            pl.BlockSpec(memory_space=pl.ANY),
            pl.BlockSpec((1, H), lambda i: (0, 0)),
            pl.BlockSpec((H, N), lambda i: (0, 0)),
            pl.BlockSpec((1, N), lambda i: (0, 0)),
            pl.BlockSpec((1, H), lambda i: (0, 0)),
            pl.BlockSpec((1, 1), lambda i: (0, 0)),
        ],
        out_specs=(
            pl.BlockSpec((R, 1), lambda i: (0, 0)),
            pl.BlockSpec((R, 1), lambda i: (0, 0)),
            pl.BlockSpec((R, 1), lambda i: (0, 0)),
        ),
        out_shape=(
            jax.ShapeDtypeStruct((R, 1), jnp.float32),
            jax.ShapeDtypeStruct((R, 1), jnp.int32),
            jax.ShapeDtypeStruct((R, 1), jnp.float32),
        ),
        scratch_shapes=[
            pltpu.VMEM((nsteps, rb), jnp.float32),
            pltpu.VMEM((H, H), jnp.float32),
            pltpu.VMEM((H, H), jnp.float32),
            pltpu.VMEM((R, H), jnp.float32),
            pltpu.SemaphoreType.DMA,
            pltpu.SemaphoreType.DMA,
        ],
    )(x2d, x2d, det_W1, det_b1.reshape(1, H), det_W2.reshape(1, H),
      det_b2.reshape(1, 1), pn_W1, pn_b1.reshape(1, H), pn_W2,
      pn_b2.reshape(1, H), id_W, id_b.reshape(1, N), g_W.reshape(1, H),
      g_b.reshape(1, 1))
    return (tim.reshape(B, _K), ids.reshape(B, _K), gains.reshape(B, _K))
